# Initial kernel scaffold; baseline (speedup 1.0000x reference)
import jax
import jax.numpy as jnp
from jax.experimental import pallas as pl


def kernel(x, edge_index, edge_weight, W1, b1, W2, b2, W3, b3, Wv1, bv1, Wv2, bv2, Wa1, ba1, Wa2, ba2):
    raise NotImplementedError("write your pallas kernel here")


# speedup vs baseline: 22.4250x; 22.4250x over previous
import functools

import jax
import jax.numpy as jnp
from jax import lax
from jax.experimental import pallas as pl
from jax.experimental.pallas import tpu as pltpu
from jax.experimental.pallas import tpu_sc as plsc

N = 10000
E = 320000
NP = 10240
TRASH = N
NC, NS, LANES = 2, 16, 16
NW = NC * NS
EPT = 10240
B = 128
NJ = EPT // B
ROWS = NP // NS
EPAD = NW * EPT

_mesh = lambda: plsc.VectorSubcoreMesh(core_axis_name="c", subcore_axis_name="s")


@functools.partial(
    pl.kernel,
    out_type=jax.ShapeDtypeStruct((NC, 2, NP, 2), jnp.float32),
    mesh=_mesh(),
    compiler_params=pltpu.CompilerParams(use_tc_tiling_on_sc=False, needs_layout_passes=False),
    scratch_types=[
        pltpu.VMEM((NJ, B), jnp.int32),
        pltpu.VMEM((NJ, B), jnp.int32),
        pltpu.VMEM((NJ, B, 2), jnp.float32),
        pltpu.VMEM_SHARED((NP, 2), jnp.float32),
        pltpu.VMEM_SHARED((NP, 2), jnp.float32),
    ],
)
def _sc_degrees(src_hbm, dst_hbm, vals_hbm, z2_hbm, out_hbm,
                idx_s, idx_d, vals_v, acc_s, acc_d):
    c = lax.axis_index("c")
    s = lax.axis_index("s")
    w = c * NS + s
    sl = pl.ds(s * ROWS, ROWS)
    pltpu.sync_copy(z2_hbm, acc_s.at[sl])
    pltpu.sync_copy(z2_hbm, acc_d.at[sl])
    plsc.subcore_barrier()
    pltpu.sync_copy(src_hbm.at[w], idx_s)
    pltpu.sync_copy(dst_hbm.at[w], idx_d)
    pltpu.sync_copy(vals_hbm.at[w], vals_v)

    def body(j, carry):
        pltpu.sync_copy(vals_v.at[j], acc_s.at[idx_s.at[j]], add=True)
        pltpu.sync_copy(vals_v.at[j], acc_d.at[idx_d.at[j]], add=True)
        return carry

    lax.fori_loop(0, NJ, body, 0)
    plsc.subcore_barrier()
    pltpu.sync_copy(acc_s.at[sl], out_hbm.at[c, 0, sl])
    pltpu.sync_copy(acc_d.at[sl], out_hbm.at[c, 1, sl])


@functools.partial(
    pl.kernel,
    out_type=(jax.ShapeDtypeStruct((NC, NP, 16), jnp.float32),
              jax.ShapeDtypeStruct((NC, NP), jnp.float32)),
    mesh=_mesh(),
    compiler_params=pltpu.CompilerParams(use_tc_tiling_on_sc=False, needs_layout_passes=False),
    scratch_types=[
        pltpu.VMEM((NJ, B), jnp.int32),
        pltpu.VMEM((NJ, B), jnp.int32),
        pltpu.VMEM((NP,), jnp.float32),
        pltpu.VMEM((B, 16), jnp.float32),
        pltpu.VMEM((B,), jnp.float32),
        pltpu.VMEM_SHARED((NP, 16), jnp.float32),
        pltpu.VMEM_SHARED((NP,), jnp.float32),
    ],
)
def _sc_prop1(src_hbm, dst_hbm, t0_hbm, nd1_hbm, z16_hbm, z1_hbm,
              p0_hbm, s_hbm,
              idx_s, idx_d, nd1t, msg, snd, acc_m, acc_1):
    c = lax.axis_index("c")
    s = lax.axis_index("s")
    w = c * NS + s
    sl = pl.ds(s * ROWS, ROWS)
    pltpu.sync_copy(z16_hbm, acc_m.at[sl])
    pltpu.sync_copy(z1_hbm, acc_1.at[sl])
    plsc.subcore_barrier()
    pltpu.sync_copy(src_hbm.at[w], idx_s)
    pltpu.sync_copy(dst_hbm.at[w], idx_d)
    pltpu.sync_copy(nd1_hbm, nd1t)

    def body(j, carry):
        pltpu.sync_copy(t0_hbm.at[idx_s.at[j]], msg)
        for g in range(B // LANES):
            d16 = idx_d[j, pl.ds(g * LANES, LANES)]
            snd[pl.ds(g * LANES, LANES)] = plsc.load_gather(nd1t, [d16])
        pltpu.sync_copy(msg, acc_m.at[idx_d.at[j]], add=True)
        pltpu.sync_copy(snd, acc_1.at[idx_s.at[j]], add=True)
        return carry

    lax.fori_loop(0, NJ, body, 0)
    plsc.subcore_barrier()
    pltpu.sync_copy(acc_m.at[sl], p0_hbm.at[c, sl])
    pltpu.sync_copy(acc_1.at[sl], s_hbm.at[c, sl])


@functools.partial(
    pl.kernel,
    out_type=jax.ShapeDtypeStruct((NC, NP, 16), jnp.float32),
    mesh=_mesh(),
    compiler_params=pltpu.CompilerParams(use_tc_tiling_on_sc=False, needs_layout_passes=False),
    scratch_types=[
        pltpu.VMEM((NJ, B), jnp.int32),
        pltpu.VMEM((NJ, B), jnp.int32),
        pltpu.VMEM((NJ, B), jnp.float32),
        pltpu.VMEM((NP,), jnp.float32),
        pltpu.VMEM((B, 16), jnp.float32),
        pltpu.VMEM_SHARED((NP, 16), jnp.float32),
    ],
)
def _sc_prop2(src_hbm, dst_hbm, ew_hbm, t1_hbm, ns2_hbm, z16_hbm,
              p1_hbm,
              idx_s, idx_d, ew_v, ns2t, msg, acc_m):
    c = lax.axis_index("c")
    s = lax.axis_index("s")
    w = c * NS + s
    sl = pl.ds(s * ROWS, ROWS)
    pltpu.sync_copy(z16_hbm, acc_m.at[sl])
    plsc.subcore_barrier()
    pltpu.sync_copy(src_hbm.at[w], idx_s)
    pltpu.sync_copy(dst_hbm.at[w], idx_d)
    pltpu.sync_copy(ew_hbm.at[w], ew_v)
    pltpu.sync_copy(ns2_hbm, ns2t)

    def body(j, carry):
        pltpu.sync_copy(t1_hbm.at[idx_s.at[j]], msg)
        for g in range(B // LANES):
            gsl = pl.ds(g * LANES, LANES)
            s16 = idx_s[j, gsl]
            a16 = ew_v[j, gsl] * plsc.load_gather(ns2t, [s16])
            e16 = lax.iota(jnp.int32, LANES) + g * LANES
            for col in range(11):
                cc = jnp.full((LANES,), col, jnp.int32)
                v = plsc.load_gather(msg, [e16, cc])
                plsc.store_scatter(msg, [e16, cc], v * a16)
        pltpu.sync_copy(msg, acc_m.at[idx_d.at[j]], add=True)
        return carry

    lax.fori_loop(0, NJ, body, 0)
    plsc.subcore_barrier()
    pltpu.sync_copy(acc_m.at[sl], p1_hbm.at[c, sl])


def _tc1_body(degp_ref, xp_ref, t0_ref, nd1_ref, ns2_ref, aux_ref):
    cs = degp_ref[0, 0, :, 0] + degp_ref[1, 0, :, 0]
    es = degp_ref[0, 0, :, 1] + degp_ref[1, 0, :, 1]
    cd = degp_ref[0, 1, :, 0] + degp_ref[1, 1, :, 0]
    ed = degp_ref[0, 1, :, 1] + degp_ref[1, 1, :, 1]
    valid = lax.broadcasted_iota(jnp.int32, (NP,), 0) < N
    ns1 = jnp.where(valid, lax.rsqrt(cs + 1.0), 0.0)
    nd1 = jnp.where(valid, lax.rsqrt(cd + 1.0), 0.0)
    ns2 = jnp.where(valid & (es > 0), lax.rsqrt(jnp.maximum(es, 1e-30)), 0.0)
    nd2 = jnp.where(valid & (ed > 0), lax.rsqrt(jnp.maximum(ed, 1e-30)), 0.0)
    t0_ref[...] = ns1[:, None] * xp_ref[...]
    nd1_ref[...] = nd1
    ns2_ref[...] = ns2
    zc = jnp.zeros((NP, 1), jnp.float32)
    aux_ref[...] = jnp.concatenate(
        [ns1[:, None], nd1[:, None], ns2[:, None], nd2[:, None],
         zc, zc, zc, zc], axis=1)


def _tc1(degp, xp):
    return pl.pallas_call(
        _tc1_body,
        out_shape=(jax.ShapeDtypeStruct((NP, 16), jnp.float32),
                   jax.ShapeDtypeStruct((NP,), jnp.float32),
                   jax.ShapeDtypeStruct((NP,), jnp.float32),
                   jax.ShapeDtypeStruct((NP, 8), jnp.float32)),
    )(degp, xp)


def _tc2_body(p0_ref, t0_ref, aux_ref, W1_ref, b1_ref, t1_ref):
    nd1 = aux_ref[:, 1:2]
    ns2 = aux_ref[:, 2:3]
    y0 = nd1 * (p0_ref[0] + p0_ref[1] + t0_ref[...])
    h1 = jnp.maximum(
        jnp.dot(y0, W1_ref[...], preferred_element_type=jnp.float32) + b1_ref[...],
        0.0)
    t1_ref[...] = ns2 * h1


def _tc2(p0, t0, aux, W1p, b1p):
    return pl.pallas_call(
        _tc2_body,
        out_shape=jax.ShapeDtypeStruct((NP, 16), jnp.float32),
    )(p0, t0, aux, W1p, b1p)


def _tc3_body(p1_ref, s_ref, aux_ref, W2_ref, b2_ref, W3_ref, b3_ref,
              Wv1_ref, bv1_ref, Wv2_ref, bv2_ref,
              Wa1_ref, ba1_ref, Wa2_ref, ba2_ref, out_ref):
    BLK = 1280

    def blk(i, acc):
        sl = pl.ds(i * BLK, BLK)
        y1 = aux_ref[sl, 3:4] * (p1_ref[0, sl, :] + p1_ref[1, sl, :])
        h2 = jnp.maximum(
            jnp.dot(y1, W2_ref[...], preferred_element_type=jnp.float32)
            + b2_ref[...], 0.0)
        cb = (aux_ref[sl, 0] * (s_ref[0, sl] + s_ref[1, sl] + aux_ref[sl, 1])
              * (1.0 / N))
        return acc + jnp.dot(cb[None, :], h2, preferred_element_type=jnp.float32)

    gbar = lax.fori_loop(0, NP // BLK, blk, jnp.zeros((1, 512), jnp.float32))
    g = jnp.dot(gbar, W3_ref[...], preferred_element_type=jnp.float32) + b3_ref[...]
    hv = jnp.maximum(
        jnp.dot(g, Wv1_ref[...], preferred_element_type=jnp.float32) + bv1_ref[...], 0.0)
    v = jnp.dot(hv, Wv2_ref[...], preferred_element_type=jnp.float32) + bv2_ref[...]
    ha = jnp.maximum(
        jnp.dot(g, Wa1_ref[...], preferred_element_type=jnp.float32) + ba1_ref[...], 0.0)
    a = jnp.dot(ha, Wa2_ref[...], preferred_element_type=jnp.float32) + ba2_ref[...]
    out_ref[...] = v + (a - jnp.mean(a))


def _tc3(p1, sp, aux, W2p, b2, W3, b3, Wv1, bv1, Wv2, bv2, Wa1, ba1, Wa2, ba2):
    return pl.pallas_call(
        _tc3_body,
        out_shape=jax.ShapeDtypeStruct((1, 1000), jnp.float32),
    )(p1, sp, aux, W2p, b2, W3, b3, Wv1, bv1, Wv2, bv2, Wa1, ba1, Wa2, ba2)


def kernel(x, edge_index, edge_weight, W1, b1, W2, b2, W3, b3,
           Wv1, bv1, Wv2, bv2, Wa1, ba1, Wa2, ba2):
    f32 = jnp.float32
    pad = EPAD - E
    src = jnp.concatenate([edge_index[0], jnp.full((pad,), TRASH, jnp.int32)])
    dst = jnp.concatenate([edge_index[1], jnp.full((pad,), TRASH, jnp.int32)])
    ewp = jnp.concatenate([edge_weight, jnp.zeros((pad,), f32)])
    src3 = src.reshape(NW, NJ, B)
    dst3 = dst.reshape(NW, NJ, B)
    ew3 = ewp.reshape(NW, NJ, B)
    vals = jnp.stack([jnp.ones((EPAD,), f32), ewp], axis=-1).reshape(NW, NJ, B, 2)

    z1 = jnp.zeros((ROWS,), f32)
    z2 = jnp.zeros((ROWS, 2), f32)
    z16 = jnp.zeros((ROWS, 16), f32)

    xp = jnp.zeros((NP, 16), f32).at[:N, :4].set(x)
    W1p = jnp.zeros((16, 16), f32).at[:4, :11].set(W1)
    b1p = jnp.zeros((1, 16), f32).at[0, :11].set(b1)
    W2p = jnp.zeros((16, 512), f32).at[:11, :].set(W2)

    degp = _sc_degrees(src3, dst3, vals, z2)
    t0, nd1, ns2, aux = _tc1(degp, xp)
    p0, sp = _sc_prop1(src3, dst3, t0, nd1, z16, z1)
    t1 = _tc2(p0, t0, aux, W1p, b1p)
    p1 = _sc_prop2(src3, dst3, ew3, t1, ns2, z16)
    return _tc3(p1, sp, aux, W2p, b2[None, :], W3, b3[None, :],
                Wv1, bv1[None, :], Wv2, bv2[None, :],
                Wa1, ba1[None, :], Wa2, ba2[None, :])
